# baseline (speedup 1.0000x reference)
import functools

import jax
import jax.numpy as jnp
from jax import lax
from jax.experimental import pallas as pl
from jax.experimental.pallas import tpu as pltpu
from jax.experimental.pallas import tpu_sc as plsc

BATCH = 16384
DIM = 64
NUM_CORES = 2
NUM_SUBCORES = 16
NUM_WORKERS = NUM_CORES * NUM_SUBCORES

SC_BATCH = 0
TC_BATCH = BATCH - SC_BATCH
B_PER_W = SC_BATCH // NUM_WORKERS


def _sc_body(idx_hbm, table_hbm, out_hbm, idx_vmem, rows_v, sem):
    wid = lax.axis_index("s") * NUM_CORES + lax.axis_index("c")
    base = wid * B_PER_W
    pltpu.sync_copy(idx_hbm.at[pl.ds(base, B_PER_W)], idx_vmem)

    def issue(c, carry):
        vec = idx_vmem[pl.ds(c * 16, 16)]
        for j in range(16):
            pltpu.async_copy(table_hbm.at[vec[j]], rows_v.at[c * 16 + j], sem)
        return carry

    lax.fori_loop(0, B_PER_W // 16, issue, 0, unroll=False)
    pltpu.make_async_copy(table_hbm.at[pl.ds(0, B_PER_W)], rows_v, sem).wait()
    pltpu.sync_copy(rows_v, out_hbm.at[pl.ds(base, B_PER_W)])


def _sc_gather(idx, weight):
    mesh = plsc.VectorSubcoreMesh(core_axis_name="c", subcore_axis_name="s")
    k = functools.partial(
        pl.kernel,
        mesh=mesh,
        out_type=jax.ShapeDtypeStruct((SC_BATCH, DIM), jnp.float32),
        scratch_types=[
            pltpu.VMEM((B_PER_W,), jnp.int32),
            pltpu.VMEM((B_PER_W, DIM), jnp.float32),
            pltpu.SemaphoreType.DMA,
        ],
    )(_sc_body)
    return k(idx, weight)


TC_ROWS_PER_STEP = 16


def _tc_body(idx_ref, *refs):
    ins, out_ref = refs[:TC_ROWS_PER_STEP], refs[TC_ROWS_PER_STEP]
    i = pl.program_id(0)
    rows = lax.broadcasted_iota(jnp.int32, (8, DIM), 0)
    for j in range(TC_ROWS_PER_STEP):
        r = idx_ref[TC_ROWS_PER_STEP * i + j] % 8
        blk = ins[j][...]
        out_ref[j, :] = jnp.sum(jnp.where(rows == r, blk, 0.0), axis=0)


def _tc_gather(idx, weight):
    grid_spec = pltpu.PrefetchScalarGridSpec(
        num_scalar_prefetch=1,
        grid=(TC_BATCH // TC_ROWS_PER_STEP,),
        in_specs=[
            pl.BlockSpec(
                (8, DIM),
                lambda i, idx_ref, j=j: (idx_ref[TC_ROWS_PER_STEP * i + j] // 8, 0),
            )
            for j in range(TC_ROWS_PER_STEP)
        ],
        out_specs=pl.BlockSpec(
            (TC_ROWS_PER_STEP, DIM), lambda i, idx_ref: (i, 0)
        ),
    )
    return pl.pallas_call(
        _tc_body,
        grid_spec=grid_spec,
        out_shape=jax.ShapeDtypeStruct((TC_BATCH, DIM), jnp.float32),
    )(idx, *([weight] * TC_ROWS_PER_STEP))


@jax.jit
def kernel(speaker_indices, weight):
    idx = speaker_indices.astype(jnp.int32)
    if SC_BATCH == 0:
        return _tc_gather(idx, weight)
    out_sc = _sc_gather(idx[:SC_BATCH], weight)
    out_tc = _tc_gather(idx[SC_BATCH:], weight)
    return jnp.concatenate([out_sc, out_tc], axis=0)


# speedup vs baseline: 2.7831x; 2.7831x over previous
import functools

import jax
import jax.numpy as jnp
from jax import lax
from jax.experimental import pallas as pl
from jax.experimental.pallas import tpu as pltpu
from jax.experimental.pallas import tpu_sc as plsc

BATCH = 16384
DIM = 64
NUM_CORES = 2
NUM_SUBCORES = 16
NUM_WORKERS = NUM_CORES * NUM_SUBCORES
B_PER_W = BATCH // NUM_WORKERS
UNROLL = 8


def _gather_body(idx_hbm, table_hbm, out_hbm, idx_vmem, rows_v, sem):
    wid = lax.axis_index("s") * NUM_CORES + lax.axis_index("c")
    base = wid * B_PER_W
    pltpu.sync_copy(idx_hbm.at[pl.ds(base, B_PER_W)], idx_vmem)

    def issue(c, carry):
        vec = idx_vmem[pl.ds(c * 16, 16)]
        for j in range(16):
            pltpu.async_copy(table_hbm.at[vec[j]], rows_v.at[c * 16 + j], sem)
        return carry

    lax.fori_loop(0, B_PER_W // 16, issue, 0, unroll=False)
    pltpu.make_async_copy(table_hbm.at[pl.ds(0, B_PER_W)], rows_v, sem).wait()
    pltpu.sync_copy(rows_v, out_hbm.at[pl.ds(base, B_PER_W)])


@jax.jit
def kernel(speaker_indices, weight):
    mesh = plsc.VectorSubcoreMesh(core_axis_name="c", subcore_axis_name="s")
    k = functools.partial(
        pl.kernel,
        mesh=mesh,
        out_type=jax.ShapeDtypeStruct((BATCH, DIM), jnp.float32),
        scratch_types=[
            pltpu.VMEM((B_PER_W,), jnp.int32),
            pltpu.VMEM((B_PER_W, DIM), jnp.float32),
            pltpu.SemaphoreType.DMA,
        ],
    )(_gather_body)
    return k(speaker_indices.astype(jnp.int32), weight)
